# baseline (speedup 1.0000x reference)
import functools

import jax
import jax.numpy as jnp
from jax import lax
from jax.experimental import pallas as pl
from jax.experimental.pallas import tpu as pltpu
from jax.experimental.pallas import tpu_sc as plsc

BOHR = 0.52917721067
N_NODES = 50000
N_EDGES = 3200000

NC = 2
NS = 16
NW = NC * NS
L = 16

NPAD = 50176
SLICE = NPAD // NS
ROWS = N_EDGES // 128
R = 40
SEC = 40
CHUNKS = ROWS // R
CPW = (CHUNKS + NW - 1) // NW


def _splat_f(v):
    return jnp.full((L,), v, jnp.float32)


def _splat_i(v):
    return jnp.full((L,), v, jnp.int32)


@functools.partial(
    pl.kernel,
    out_type=jax.ShapeDtypeStruct((NC * NPAD,), jnp.float32),
    mesh=plsc.VectorSubcoreMesh(
        core_axis_name="c", subcore_axis_name="s",
        num_cores=NC, num_subcores=NS),
    compiler_params=pltpu.CompilerParams(needs_layout_passes=False),
    scratch_types=[
        pltpu.VMEM((NPAD,), jnp.float32),
        pltpu.VMEM((NPAD,), jnp.float32),
        pltpu.VMEM((R, 128), jnp.int32),
        pltpu.VMEM((R, 128), jnp.int32),
        pltpu.VMEM((R, 128), jnp.float32),
        pltpu.VMEM((R, 128), jnp.float32),
        pltpu.VMEM((R * 128,), jnp.float32),
        pltpu.VMEM((4 * L,), jnp.float32),
        pltpu.VMEM((96,), jnp.float32),
        pltpu.VMEM((96,), jnp.float32),
        pltpu.SemaphoreType.DMA,
        pltpu.SemaphoreType.DMA,
        pltpu.VMEM_SHARED((NPAD,), jnp.float32),
    ],
)
def _coulomb_sc(sp_hbm, ch_hbm, src_hbm, dst_hbm, dist_hbm, sw_hbm,
                par_hbm, vdw_hbm, val_hbm, out_hbm,
                tabQ_v, tabA_v, src_v, dst_v, dist_v, sw_v, ep_v,
                par_v, vdw_v, val_v, semI, semS, acc):
    c = lax.axis_index("c")
    s = lax.axis_index("s")
    wid = s * NC + c
    nbase = s * SLICE

    pltpu.sync_copy(par_hbm, par_v)
    pltpu.sync_copy(vdw_hbm, vdw_v)
    pltpu.sync_copy(val_hbm, val_v)

    rb0 = wid * R
    pltpu.async_copy(src_hbm.at[pl.ds(rb0, R)], src_v, semI)
    pltpu.async_copy(dst_hbm.at[pl.ds(rb0, R)], dst_v, semI)
    pltpu.async_copy(dist_hbm.at[pl.ds(rb0, R)], dist_v, semI)
    pltpu.async_copy(sw_hbm.at[pl.ds(rb0, R)], sw_v, semI)

    rv = par_v[pl.ds(0 * L, L)]
    sv = par_v[pl.ds(1 * L, L)]
    cscv = par_v[pl.ds(2 * L, L)]
    acpv = par_v[pl.ds(3 * L, L)]
    one = _splat_f(1.0)
    m7 = _splat_i(7)
    mm8 = _splat_i(-8)

    pltpu.sync_copy(ch_hbm.at[pl.ds(nbase, SLICE)], ep_v.at[pl.ds(0, SLICE)])

    @plsc.parallel_loop(0, SLICE // L)
    def q_body(i):
        sl = pl.ds(i * L, L)
        tabQ_v[sl] = ep_v[sl] * cscv
    pltpu.sync_copy(tabQ_v.at[pl.ds(0, SLICE)], acc.at[pl.ds(nbase, SLICE)])
    plsc.subcore_barrier()
    rdQ = pltpu.async_copy(acc, tabQ_v, semS)

    pltpu.sync_copy(sp_hbm.at[pl.ds(nbase, SLICE)], ep_v.at[pl.ds(0, SLICE)])

    @plsc.parallel_loop(0, SLICE // L)
    def a_body(i):
        sl = pl.ds(i * L, L)
        sp = ep_v[sl].astype(jnp.int32)
        v = plsc.load_gather(vdw_v, [sp])
        z = plsc.load_gather(val_v, [sp])
        a = acpv / v
        bits = ((lax.bitcast_convert_type(a, jnp.int32) & mm8)
                | (z.astype(jnp.int32) - 1))
        tabA_v[sl] = plsc.bitcast(bits, jnp.float32)
    rdQ.wait()
    plsc.subcore_barrier()
    pltpu.sync_copy(tabA_v.at[pl.ds(0, SLICE)], acc.at[pl.ds(nbase, SLICE)])
    plsc.subcore_barrier()
    rdA = pltpu.async_copy(acc, tabA_v, semS)

    def _zb(i, carry):
        ep_v[pl.ds(i * L, L)] = _splat_f(0.0)
        return carry
    lax.fori_loop(0, SLICE // L, _zb, 0)
    rdA.wait()
    plsc.subcore_barrier()
    pltpu.sync_copy(ep_v.at[pl.ds(0, SLICE)], acc.at[pl.ds(nbase, SLICE)])
    plsc.subcore_barrier()

    def _drain_scatters():
        for j in range(R):
            pltpu.make_async_copy(ep_v.at[pl.ds(j * 128, 128)],
                                  acc.at[src_v.at[j]], semS).wait()

    def chunk_body(ci, carry):
        g = wid + ci * NW

        @pl.when(g < CHUNKS)
        def _():
            rb = g * R

            @pl.when(ci > 0)
            def _():
                pltpu.async_copy(dst_hbm.at[pl.ds(rb, R)], dst_v, semI)
                pltpu.async_copy(dist_hbm.at[pl.ds(rb, R)], dist_v, semI)
                pltpu.async_copy(sw_hbm.at[pl.ds(rb, R)], sw_v, semI)
                _drain_scatters()
                pltpu.async_copy(src_hbm.at[pl.ds(rb, R)], src_v, semI)

            pltpu.make_async_copy(dst_hbm.at[pl.ds(rb, R)], dst_v, semI).wait()
            pltpu.make_async_copy(dist_hbm.at[pl.ds(rb, R)], dist_v, semI).wait()
            pltpu.make_async_copy(sw_hbm.at[pl.ds(rb, R)], sw_v, semI).wait()
            pltpu.make_async_copy(src_hbm.at[pl.ds(rb, R)], src_v, semI).wait()

            for sec in range(R // SEC):
                @plsc.parallel_loop(sec * SEC, (sec + 1) * SEC)
                def row_body(j):
                    for gg in range(128 // L):
                        sl = pl.ds(gg * L, L)
                        isc = src_v[j, sl]
                        idc = dst_v[j, sl]
                        qi = plsc.load_gather(tabQ_v, [isc])
                        qj = plsc.load_gather(tabQ_v, [idc])
                        ai = plsc.load_gather(tabA_v, [isc])
                        aj = plsc.load_gather(tabA_v, [idc])
                        pi = plsc.bitcast(ai, jnp.int32)
                        pj = plsc.bitcast(aj, jnp.int32)
                        zi = (pi & m7).astype(jnp.float32) + one
                        zj = (pj & m7).astype(jnp.float32) + one
                        d = dist_v[j, sl]
                        dai = d * ai
                        daj = d * aj
                        eAi = jnp.exp(dai)
                        eAj = jnp.exp(daj)
                        eBi = jnp.exp(dai * rv)
                        eBj = jnp.exp(daj * rv)
                        ui = zi - qi
                        uj = zj - qj
                        ti = ui * (eBi - one)
                        tj = uj * (eBj - one)
                        core = (zj * ui * eAi + zi * uj * eAj
                                + ti * tj - ui * uj + qi * qj)
                        ep_v[pl.ds(j * 128 + gg * L, L)] = (
                            core * sw_v[j, sl] * sv / d)
                for j in range(sec * SEC, (sec + 1) * SEC):
                    pltpu.async_copy(ep_v.at[pl.ds(j * 128, 128)],
                                     acc.at[src_v.at[j]], semS, add=True)
        return carry

    lax.fori_loop(0, CPW, chunk_body, 0)
    _drain_scatters()

    plsc.subcore_barrier()
    pltpu.sync_copy(acc.at[pl.ds(nbase, SLICE)], ep_v.at[pl.ds(0, SLICE)])
    pltpu.sync_copy(ep_v.at[pl.ds(0, SLICE)],
                    out_hbm.at[pl.ds(c * NPAD + nbase, SLICE)])


def kernel(species, edge_src, edge_dst, distances, switch, charges,
           vdw_radii, d3_vdw_radii, valence_electrons,
           cpA, cpB, gamma, charge_scale, scale):
    N = species.shape[0]
    cpA_ = jnp.abs(cpA).astype(jnp.float32)
    cpB_ = jnp.abs(cpB).astype(jnp.float32)

    spf = jnp.pad(species.astype(jnp.float32), (0, NPAD - N))
    chf = jnp.pad(charges.astype(jnp.float32), (0, NPAD - N))
    vdw96 = jnp.pad(vdw_radii.astype(jnp.float32), (0, 96 - vdw_radii.shape[0]),
                    constant_values=1.0)
    val96 = jnp.pad(valence_electrons.astype(jnp.float32),
                    (0, 96 - valence_electrons.shape[0]), constant_values=1.0)
    params = jnp.concatenate([
        jnp.full((L,), cpB_ / cpA_, jnp.float32),
        jnp.full((L,), 0.5 * BOHR * jnp.abs(scale), jnp.float32),
        jnp.full((L,), jnp.abs(charge_scale), jnp.float32),
        jnp.full((L,), -cpA_ / BOHR, jnp.float32),
    ])

    src2 = edge_src.astype(jnp.int32).reshape(ROWS, 128)
    dst2 = edge_dst.astype(jnp.int32).reshape(ROWS, 128)
    dist2 = distances.astype(jnp.float32).reshape(ROWS, 128)
    sw2 = switch.astype(jnp.float32).reshape(ROWS, 128)

    out = _coulomb_sc(spf, chf, src2, dst2, dist2, sw2, params, vdw96, val96)
    return (out[:NPAD] + out[NPAD:])[:N]


# speedup vs baseline: 1.0248x; 1.0248x over previous
import functools

import jax
import jax.numpy as jnp
from jax import lax
from jax.experimental import pallas as pl
from jax.experimental.pallas import tpu as pltpu
from jax.experimental.pallas import tpu_sc as plsc

BOHR = 0.52917721067
N_NODES = 50000
N_EDGES = 3200000

NC = 2
NS = 16
NW = NC * NS
L = 16

NPAD = 50176
SLICE = NPAD // NS
ROWS = N_EDGES // 128
R = 40
SEC = 20
CHUNKS = ROWS // R
CPW = (CHUNKS + NW - 1) // NW


def _splat_f(v):
    return jnp.full((L,), v, jnp.float32)


def _splat_i(v):
    return jnp.full((L,), v, jnp.int32)


@functools.partial(
    pl.kernel,
    out_type=jax.ShapeDtypeStruct((NC * NPAD,), jnp.float32),
    mesh=plsc.VectorSubcoreMesh(
        core_axis_name="c", subcore_axis_name="s",
        num_cores=NC, num_subcores=NS),
    compiler_params=pltpu.CompilerParams(needs_layout_passes=False),
    scratch_types=[
        pltpu.VMEM((NPAD,), jnp.float32),
        pltpu.VMEM((NPAD,), jnp.float32),
        pltpu.VMEM((R, 128), jnp.int32),
        pltpu.VMEM((R, 128), jnp.int32),
        pltpu.VMEM((R, 128), jnp.float32),
        pltpu.VMEM((R, 128), jnp.float32),
        pltpu.VMEM((R * 128,), jnp.float32),
        pltpu.VMEM((4 * L,), jnp.float32),
        pltpu.VMEM((96,), jnp.float32),
        pltpu.VMEM((96,), jnp.float32),
        pltpu.SemaphoreType.DMA,
        pltpu.SemaphoreType.DMA,
        pltpu.VMEM_SHARED((NPAD,), jnp.float32),
    ],
)
def _coulomb_sc(sp_hbm, ch_hbm, src_hbm, dst_hbm, dist_hbm, sw_hbm,
                par_hbm, vdw_hbm, val_hbm, out_hbm,
                tabQ_v, tabA_v, src_v, dst_v, dist_v, sw_v, ep_v,
                par_v, vdw_v, val_v, semI, semS, acc):
    c = lax.axis_index("c")
    s = lax.axis_index("s")
    wid = s * NC + c
    nbase = s * SLICE

    pltpu.sync_copy(par_hbm, par_v)
    pltpu.sync_copy(vdw_hbm, vdw_v)
    pltpu.sync_copy(val_hbm, val_v)

    rb0 = wid * R
    pltpu.async_copy(src_hbm.at[pl.ds(rb0, R)], src_v, semI)
    pltpu.async_copy(dst_hbm.at[pl.ds(rb0, R)], dst_v, semI)
    pltpu.async_copy(dist_hbm.at[pl.ds(rb0, R)], dist_v, semI)
    pltpu.async_copy(sw_hbm.at[pl.ds(rb0, R)], sw_v, semI)

    rv = par_v[pl.ds(0 * L, L)]
    sv = par_v[pl.ds(1 * L, L)]
    cscv = par_v[pl.ds(2 * L, L)]
    acpv = par_v[pl.ds(3 * L, L)]
    one = _splat_f(1.0)
    m7 = _splat_i(7)
    mm8 = _splat_i(-8)

    pltpu.sync_copy(ch_hbm.at[pl.ds(nbase, SLICE)], ep_v.at[pl.ds(0, SLICE)])

    @plsc.parallel_loop(0, SLICE // L)
    def q_body(i):
        sl = pl.ds(i * L, L)
        tabQ_v[sl] = ep_v[sl] * cscv
    pltpu.sync_copy(tabQ_v.at[pl.ds(0, SLICE)], acc.at[pl.ds(nbase, SLICE)])
    plsc.subcore_barrier()
    rdQ = pltpu.async_copy(acc, tabQ_v, semS)

    pltpu.sync_copy(sp_hbm.at[pl.ds(nbase, SLICE)], ep_v.at[pl.ds(0, SLICE)])

    @plsc.parallel_loop(0, SLICE // L)
    def a_body(i):
        sl = pl.ds(i * L, L)
        sp = ep_v[sl].astype(jnp.int32)
        v = plsc.load_gather(vdw_v, [sp])
        z = plsc.load_gather(val_v, [sp])
        a = acpv / v
        bits = ((lax.bitcast_convert_type(a, jnp.int32) & mm8)
                | (z.astype(jnp.int32) - 1))
        tabA_v[sl] = plsc.bitcast(bits, jnp.float32)
    rdQ.wait()
    plsc.subcore_barrier()
    pltpu.sync_copy(tabA_v.at[pl.ds(0, SLICE)], acc.at[pl.ds(nbase, SLICE)])
    plsc.subcore_barrier()
    rdA = pltpu.async_copy(acc, tabA_v, semS)

    def _zb(i, carry):
        ep_v[pl.ds(i * L, L)] = _splat_f(0.0)
        return carry
    lax.fori_loop(0, SLICE // L, _zb, 0)
    rdA.wait()
    plsc.subcore_barrier()
    pltpu.sync_copy(ep_v.at[pl.ds(0, SLICE)], acc.at[pl.ds(nbase, SLICE)])
    plsc.subcore_barrier()

    def _drain_scatters():
        for j in range(R):
            pltpu.make_async_copy(ep_v.at[pl.ds(j * 128, 128)],
                                  acc.at[src_v.at[j]], semS).wait()

    def chunk_body(ci, carry):
        g = wid + ci * NW

        @pl.when(g < CHUNKS)
        def _():
            rb = g * R

            @pl.when(ci > 0)
            def _():
                pltpu.async_copy(dst_hbm.at[pl.ds(rb, R)], dst_v, semI)
                pltpu.async_copy(dist_hbm.at[pl.ds(rb, R)], dist_v, semI)
                pltpu.async_copy(sw_hbm.at[pl.ds(rb, R)], sw_v, semI)
                _drain_scatters()
                pltpu.async_copy(src_hbm.at[pl.ds(rb, R)], src_v, semI)

            pltpu.make_async_copy(dst_hbm.at[pl.ds(rb, R)], dst_v, semI).wait()
            pltpu.make_async_copy(dist_hbm.at[pl.ds(rb, R)], dist_v, semI).wait()
            pltpu.make_async_copy(sw_hbm.at[pl.ds(rb, R)], sw_v, semI).wait()
            pltpu.make_async_copy(src_hbm.at[pl.ds(rb, R)], src_v, semI).wait()

            for sec in range(R // SEC):
                @plsc.parallel_loop(sec * SEC, (sec + 1) * SEC)
                def row_body(j):
                    for gg in range(128 // L):
                        sl = pl.ds(gg * L, L)
                        isc = src_v[j, sl]
                        idc = dst_v[j, sl]
                        qi = plsc.load_gather(tabQ_v, [isc])
                        qj = plsc.load_gather(tabQ_v, [idc])
                        ai = plsc.load_gather(tabA_v, [isc])
                        aj = plsc.load_gather(tabA_v, [idc])
                        pi = plsc.bitcast(ai, jnp.int32)
                        pj = plsc.bitcast(aj, jnp.int32)
                        zi = (pi & m7).astype(jnp.float32) + one
                        zj = (pj & m7).astype(jnp.float32) + one
                        d = dist_v[j, sl]
                        dai = d * ai
                        daj = d * aj
                        eAi = jnp.exp(dai)
                        eAj = jnp.exp(daj)
                        eBi = jnp.exp(dai * rv)
                        eBj = jnp.exp(daj * rv)
                        ui = zi - qi
                        uj = zj - qj
                        ti = ui * (eBi - one)
                        tj = uj * (eBj - one)
                        core = (zj * ui * eAi + zi * uj * eAj
                                + ti * tj - ui * uj + qi * qj)
                        ep_v[pl.ds(j * 128 + gg * L, L)] = (
                            core * sw_v[j, sl] * sv / d)
                for j in range(sec * SEC, (sec + 1) * SEC):
                    pltpu.async_copy(ep_v.at[pl.ds(j * 128, 128)],
                                     acc.at[src_v.at[j]], semS, add=True)
        return carry

    lax.fori_loop(0, CPW, chunk_body, 0)
    _drain_scatters()

    plsc.subcore_barrier()
    pltpu.sync_copy(acc.at[pl.ds(nbase, SLICE)], ep_v.at[pl.ds(0, SLICE)])
    pltpu.sync_copy(ep_v.at[pl.ds(0, SLICE)],
                    out_hbm.at[pl.ds(c * NPAD + nbase, SLICE)])


def kernel(species, edge_src, edge_dst, distances, switch, charges,
           vdw_radii, d3_vdw_radii, valence_electrons,
           cpA, cpB, gamma, charge_scale, scale):
    N = species.shape[0]
    cpA_ = jnp.abs(cpA).astype(jnp.float32)
    cpB_ = jnp.abs(cpB).astype(jnp.float32)

    spf = jnp.pad(species.astype(jnp.float32), (0, NPAD - N))
    chf = jnp.pad(charges.astype(jnp.float32), (0, NPAD - N))
    vdw96 = jnp.pad(vdw_radii.astype(jnp.float32), (0, 96 - vdw_radii.shape[0]),
                    constant_values=1.0)
    val96 = jnp.pad(valence_electrons.astype(jnp.float32),
                    (0, 96 - valence_electrons.shape[0]), constant_values=1.0)
    params = jnp.concatenate([
        jnp.full((L,), cpB_ / cpA_, jnp.float32),
        jnp.full((L,), 0.5 * BOHR * jnp.abs(scale), jnp.float32),
        jnp.full((L,), jnp.abs(charge_scale), jnp.float32),
        jnp.full((L,), -cpA_ / BOHR, jnp.float32),
    ])

    src2 = edge_src.astype(jnp.int32).reshape(ROWS, 128)
    dst2 = edge_dst.astype(jnp.int32).reshape(ROWS, 128)
    dist2 = distances.astype(jnp.float32).reshape(ROWS, 128)
    sw2 = switch.astype(jnp.float32).reshape(ROWS, 128)

    out = _coulomb_sc(spf, chf, src2, dst2, dist2, sw2, params, vdw96, val96)
    return (out[:NPAD] + out[NPAD:])[:N]


# speedup vs baseline: 1.0555x; 1.0299x over previous
import functools

import jax
import jax.numpy as jnp
from jax import lax
from jax.experimental import pallas as pl
from jax.experimental.pallas import tpu as pltpu
from jax.experimental.pallas import tpu_sc as plsc

BOHR = 0.52917721067
N_NODES = 50000
N_EDGES = 3200000

NC = 2
NS = 16
NW = NC * NS
L = 16

NPAD = 50176
SLICE = NPAD // NS
NTAIL = N_NODES - (NS - 1) * SLICE
ROWS = N_EDGES // 128
R = 40
SEC = 20
CHUNKS = ROWS // R
CPW = (CHUNKS + NW - 1) // NW


def _splat_f(v):
    return jnp.full((L,), v, jnp.float32)


def _splat_i(v):
    return jnp.full((L,), v, jnp.int32)


@functools.partial(
    pl.kernel,
    out_type=jax.ShapeDtypeStruct((NC * NPAD,), jnp.float32),
    mesh=plsc.VectorSubcoreMesh(
        core_axis_name="c", subcore_axis_name="s",
        num_cores=NC, num_subcores=NS),
    compiler_params=pltpu.CompilerParams(needs_layout_passes=False),
    scratch_types=[
        pltpu.VMEM((NPAD,), jnp.float32),
        pltpu.VMEM((NPAD,), jnp.float32),
        pltpu.VMEM((R, 128), jnp.int32),
        pltpu.VMEM((R * 128,), jnp.int32),
        pltpu.VMEM((R, 128), jnp.float32),
        pltpu.VMEM((R, 128), jnp.float32),
        pltpu.VMEM((R * 128,), jnp.float32),
        pltpu.VMEM((4 * L,), jnp.float32),
        pltpu.VMEM((96,), jnp.float32),
        pltpu.VMEM((96,), jnp.float32),
        pltpu.SemaphoreType.DMA,
        pltpu.SemaphoreType.DMA,
        pltpu.VMEM_SHARED((NPAD,), jnp.float32),
    ],
)
def _coulomb_sc(sp_hbm, ch_hbm, src_hbm, dst_hbm, dist_hbm, sw_hbm,
                par_hbm, vdw_hbm, val_hbm, out_hbm,
                tabQ_v, tabA_v, src_v, dst_v, dist_v, sw_v, ep_v,
                par_v, vdw_v, val_v, semI, semS, acc):
    c = lax.axis_index("c")
    s = lax.axis_index("s")
    wid = s * NC + c
    nbase = s * SLICE

    pltpu.sync_copy(par_hbm, par_v)
    pltpu.sync_copy(vdw_hbm, vdw_v)
    pltpu.sync_copy(val_hbm, val_v)

    rb0 = wid * R
    pltpu.async_copy(src_hbm.at[pl.ds(rb0, R)], src_v, semI)
    pltpu.async_copy(dist_hbm.at[pl.ds(rb0, R)], dist_v, semI)
    pltpu.async_copy(sw_hbm.at[pl.ds(rb0, R)], sw_v, semI)

    rv = par_v[pl.ds(0 * L, L)]
    sv = par_v[pl.ds(1 * L, L)]
    cscv = par_v[pl.ds(2 * L, L)]
    acpv = par_v[pl.ds(3 * L, L)]
    one = _splat_f(1.0)
    m7 = _splat_i(7)
    mm8 = _splat_i(-8)

    @pl.when(s < NS - 1)
    def _():
        pltpu.sync_copy(ch_hbm.at[pl.ds(nbase, SLICE)],
                        ep_v.at[pl.ds(0, SLICE)])

    @pl.when(s == NS - 1)
    def _():
        pltpu.sync_copy(ch_hbm.at[pl.ds(nbase, NTAIL)],
                        ep_v.at[pl.ds(0, NTAIL)])

        def _zt(i, carry):
            ep_v[pl.ds(NTAIL + i * L, L)] = _splat_f(0.0)
            return carry
        lax.fori_loop(0, (SLICE - NTAIL) // L, _zt, 0)

    @plsc.parallel_loop(0, SLICE // L)
    def q_body(i):
        sl = pl.ds(i * L, L)
        tabQ_v[sl] = ep_v[sl] * cscv
    pltpu.sync_copy(tabQ_v.at[pl.ds(0, SLICE)], acc.at[pl.ds(nbase, SLICE)])
    plsc.subcore_barrier()
    rdQ = pltpu.async_copy(acc, tabQ_v, semS)

    @pl.when(s < NS - 1)
    def _():
        pltpu.sync_copy(sp_hbm.at[pl.ds(nbase, SLICE)],
                        dst_v.at[pl.ds(0, SLICE)])

    @pl.when(s == NS - 1)
    def _():
        pltpu.sync_copy(sp_hbm.at[pl.ds(nbase, NTAIL)],
                        dst_v.at[pl.ds(0, NTAIL)])

        def _zt(i, carry):
            dst_v[pl.ds(NTAIL + i * L, L)] = _splat_i(0)
            return carry
        lax.fori_loop(0, (SLICE - NTAIL) // L, _zt, 0)

    @plsc.parallel_loop(0, SLICE // L)
    def a_body(i):
        sl = pl.ds(i * L, L)
        sp = dst_v[sl]
        v = plsc.load_gather(vdw_v, [sp])
        z = plsc.load_gather(val_v, [sp])
        a = acpv / v
        bits = ((lax.bitcast_convert_type(a, jnp.int32) & mm8)
                | (z.astype(jnp.int32) - 1))
        tabA_v[sl] = plsc.bitcast(bits, jnp.float32)
    pltpu.async_copy(dst_hbm.at[pl.ds(rb0 * 128, R * 128)], dst_v, semI)
    rdQ.wait()
    plsc.subcore_barrier()
    pltpu.sync_copy(tabA_v.at[pl.ds(0, SLICE)], acc.at[pl.ds(nbase, SLICE)])
    plsc.subcore_barrier()
    rdA = pltpu.async_copy(acc, tabA_v, semS)

    def _zb(i, carry):
        ep_v[pl.ds(i * L, L)] = _splat_f(0.0)
        return carry
    lax.fori_loop(0, SLICE // L, _zb, 0)
    rdA.wait()
    plsc.subcore_barrier()
    pltpu.sync_copy(ep_v.at[pl.ds(0, SLICE)], acc.at[pl.ds(nbase, SLICE)])
    plsc.subcore_barrier()

    def _drain_scatters():
        for j in range(R):
            pltpu.make_async_copy(ep_v.at[pl.ds(j * 128, 128)],
                                  acc.at[src_v.at[j]], semS).wait()

    def chunk_body(ci, carry):
        g = wid + ci * NW

        @pl.when(g < CHUNKS)
        def _():
            rb = g * R

            @pl.when(ci > 0)
            def _():
                pltpu.async_copy(dst_hbm.at[pl.ds(rb * 128, R * 128)],
                                 dst_v, semI)
                pltpu.async_copy(dist_hbm.at[pl.ds(rb, R)], dist_v, semI)
                pltpu.async_copy(sw_hbm.at[pl.ds(rb, R)], sw_v, semI)
                _drain_scatters()
                pltpu.async_copy(src_hbm.at[pl.ds(rb, R)], src_v, semI)

            pltpu.make_async_copy(dst_hbm.at[pl.ds(rb * 128, R * 128)],
                                  dst_v, semI).wait()
            pltpu.make_async_copy(dist_hbm.at[pl.ds(rb, R)], dist_v, semI).wait()
            pltpu.make_async_copy(sw_hbm.at[pl.ds(rb, R)], sw_v, semI).wait()
            pltpu.make_async_copy(src_hbm.at[pl.ds(rb, R)], src_v, semI).wait()

            for sec in range(R // SEC):
                @plsc.parallel_loop(sec * SEC, (sec + 1) * SEC)
                def row_body(j):
                    for gg in range(128 // L):
                        sl = pl.ds(gg * L, L)
                        isc = src_v[j, sl]
                        idc = dst_v[pl.ds(j * 128 + gg * L, L)]
                        qi = plsc.load_gather(tabQ_v, [isc])
                        qj = plsc.load_gather(tabQ_v, [idc])
                        ai = plsc.load_gather(tabA_v, [isc])
                        aj = plsc.load_gather(tabA_v, [idc])
                        pi = plsc.bitcast(ai, jnp.int32)
                        pj = plsc.bitcast(aj, jnp.int32)
                        zi = (pi & m7).astype(jnp.float32) + one
                        zj = (pj & m7).astype(jnp.float32) + one
                        d = dist_v[j, sl]
                        dai = d * ai
                        daj = d * aj
                        eAi = jnp.exp(dai)
                        eAj = jnp.exp(daj)
                        eBi = jnp.exp(dai * rv)
                        eBj = jnp.exp(daj * rv)
                        ui = zi - qi
                        uj = zj - qj
                        ti = ui * (eBi - one)
                        tj = uj * (eBj - one)
                        core = (zj * ui * eAi + zi * uj * eAj
                                + ti * tj - ui * uj + qi * qj)
                        ep_v[pl.ds(j * 128 + gg * L, L)] = (
                            core * sw_v[j, sl] * sv / d)
                for j in range(sec * SEC, (sec + 1) * SEC):
                    pltpu.async_copy(ep_v.at[pl.ds(j * 128, 128)],
                                     acc.at[src_v.at[j]], semS, add=True)
        return carry

    lax.fori_loop(0, CPW, chunk_body, 0)
    _drain_scatters()

    plsc.subcore_barrier()
    pltpu.sync_copy(acc.at[pl.ds(nbase, SLICE)], ep_v.at[pl.ds(0, SLICE)])
    pltpu.sync_copy(ep_v.at[pl.ds(0, SLICE)],
                    out_hbm.at[pl.ds(c * NPAD + nbase, SLICE)])


def kernel(species, edge_src, edge_dst, distances, switch, charges,
           vdw_radii, d3_vdw_radii, valence_electrons,
           cpA, cpB, gamma, charge_scale, scale):
    N = species.shape[0]
    cpA_ = jnp.abs(cpA).astype(jnp.float32)
    cpB_ = jnp.abs(cpB).astype(jnp.float32)

    spi = species.astype(jnp.int32)
    chf = charges.astype(jnp.float32)
    vdw96 = jnp.pad(vdw_radii.astype(jnp.float32), (0, 96 - vdw_radii.shape[0]),
                    constant_values=1.0)
    val96 = jnp.pad(valence_electrons.astype(jnp.float32),
                    (0, 96 - valence_electrons.shape[0]), constant_values=1.0)
    params = jnp.concatenate([
        jnp.full((L,), cpB_ / cpA_, jnp.float32),
        jnp.full((L,), 0.5 * BOHR * jnp.abs(scale), jnp.float32),
        jnp.full((L,), jnp.abs(charge_scale), jnp.float32),
        jnp.full((L,), -cpA_ / BOHR, jnp.float32),
    ])

    src2 = edge_src.astype(jnp.int32).reshape(ROWS, 128)
    dst1 = edge_dst.astype(jnp.int32)
    dist2 = distances.astype(jnp.float32).reshape(ROWS, 128)
    sw2 = switch.astype(jnp.float32).reshape(ROWS, 128)

    out = _coulomb_sc(spi, chf, src2, dst1, dist2, sw2, params, vdw96, val96)
    return (out[:NPAD] + out[NPAD:])[:N]


# speedup vs baseline: 1.0866x; 1.0295x over previous
import functools

import jax
import jax.numpy as jnp
from jax import lax
from jax.experimental import pallas as pl
from jax.experimental.pallas import tpu as pltpu
from jax.experimental.pallas import tpu_sc as plsc

BOHR = 0.52917721067
N_NODES = 50000
N_EDGES = 3200000

NC = 2
NS = 16
NW = NC * NS
L = 16

NPAD = 50176
SLICE = NPAD // NS
NTAIL = N_NODES - (NS - 1) * SLICE
ROWS = N_EDGES // 128
R = 40
SEC = 20
CHUNKS = ROWS // R
CPW = (CHUNKS + NW - 1) // NW


def _splat_f(v):
    return jnp.full((L,), v, jnp.float32)


def _splat_i(v):
    return jnp.full((L,), v, jnp.int32)


@functools.partial(
    pl.kernel,
    out_type=jax.ShapeDtypeStruct((NC * NPAD,), jnp.float32),
    mesh=plsc.VectorSubcoreMesh(
        core_axis_name="c", subcore_axis_name="s",
        num_cores=NC, num_subcores=NS),
    compiler_params=pltpu.CompilerParams(needs_layout_passes=False),
    scratch_types=[
        pltpu.VMEM((NPAD,), jnp.float32),
        pltpu.VMEM((NPAD,), jnp.float32),
        pltpu.VMEM((R, 128), jnp.int32),
        pltpu.VMEM((R * 128,), jnp.int32),
        pltpu.VMEM((R, 128), jnp.float32),
        pltpu.VMEM((R, 128), jnp.float32),
        pltpu.VMEM((R * 128,), jnp.float32),
        pltpu.VMEM((4 * L,), jnp.float32),
        pltpu.VMEM((96,), jnp.float32),
        pltpu.VMEM((96,), jnp.float32),
        pltpu.SemaphoreType.DMA,
        pltpu.SemaphoreType.DMA,
        pltpu.VMEM_SHARED((NPAD,), jnp.float32),
    ],
)
def _coulomb_sc(sp_hbm, ch_hbm, src_hbm, dst_hbm, dist_hbm, sw_hbm,
                par_hbm, vdw_hbm, val_hbm, out_hbm,
                tabQ_v, tabA_v, src_v, dst_v, dist_v, sw_v, ep_v,
                par_v, vdw_v, val_v, semI, semS, acc):
    c = lax.axis_index("c")
    s = lax.axis_index("s")
    wid = s * NC + c
    nbase = s * SLICE

    pltpu.sync_copy(par_hbm, par_v)
    pltpu.sync_copy(vdw_hbm, vdw_v)
    pltpu.sync_copy(val_hbm, val_v)

    rb0 = wid * R
    pltpu.async_copy(src_hbm.at[pl.ds(rb0, R)], src_v, semI)
    pltpu.async_copy(dist_hbm.at[pl.ds(rb0, R)], dist_v, semI)
    pltpu.async_copy(sw_hbm.at[pl.ds(rb0, R)], sw_v, semI)

    rv = par_v[pl.ds(0 * L, L)]
    sv = par_v[pl.ds(1 * L, L)]
    cscv = par_v[pl.ds(2 * L, L)]
    acpv = par_v[pl.ds(3 * L, L)]
    one = _splat_f(1.0)
    m7 = _splat_i(7)
    mm8 = _splat_i(-8)

    @pl.when(s < NS - 1)
    def _():
        pltpu.sync_copy(ch_hbm.at[pl.ds(nbase, SLICE)],
                        ep_v.at[pl.ds(0, SLICE)])

    @pl.when(s == NS - 1)
    def _():
        pltpu.sync_copy(ch_hbm.at[pl.ds(nbase, NTAIL)],
                        ep_v.at[pl.ds(0, NTAIL)])

        def _zt(i, carry):
            ep_v[pl.ds(NTAIL + i * L, L)] = _splat_f(0.0)
            return carry
        lax.fori_loop(0, (SLICE - NTAIL) // L, _zt, 0)

    @plsc.parallel_loop(0, SLICE // L)
    def q_body(i):
        sl = pl.ds(i * L, L)
        tabQ_v[sl] = ep_v[sl] * cscv
    pltpu.sync_copy(tabQ_v.at[pl.ds(0, SLICE)], acc.at[pl.ds(nbase, SLICE)])
    plsc.subcore_barrier()
    rdQ = pltpu.async_copy(acc, tabQ_v, semS)

    @pl.when(s < NS - 1)
    def _():
        pltpu.sync_copy(sp_hbm.at[pl.ds(nbase, SLICE)],
                        dst_v.at[pl.ds(0, SLICE)])

    @pl.when(s == NS - 1)
    def _():
        pltpu.sync_copy(sp_hbm.at[pl.ds(nbase, NTAIL)],
                        dst_v.at[pl.ds(0, NTAIL)])

        def _zt(i, carry):
            dst_v[pl.ds(NTAIL + i * L, L)] = _splat_i(0)
            return carry
        lax.fori_loop(0, (SLICE - NTAIL) // L, _zt, 0)

    @plsc.parallel_loop(0, SLICE // L)
    def a_body(i):
        sl = pl.ds(i * L, L)
        sp = dst_v[sl]
        v = plsc.load_gather(vdw_v, [sp])
        z = plsc.load_gather(val_v, [sp])
        a = acpv / v
        bits = ((lax.bitcast_convert_type(a, jnp.int32) & mm8)
                | (z.astype(jnp.int32) - 1))
        tabA_v[sl] = plsc.bitcast(bits, jnp.float32)
    pltpu.async_copy(dst_hbm.at[pl.ds(rb0 * 128, R * 128)], dst_v, semI)
    rdQ.wait()
    plsc.subcore_barrier()
    pltpu.sync_copy(tabA_v.at[pl.ds(0, SLICE)], acc.at[pl.ds(nbase, SLICE)])
    plsc.subcore_barrier()
    rdA = pltpu.async_copy(acc, tabA_v, semS)

    def _zb(i, carry):
        ep_v[pl.ds(i * L, L)] = _splat_f(0.0)
        return carry
    lax.fori_loop(0, SLICE // L, _zb, 0)
    rdA.wait()
    plsc.subcore_barrier()
    pltpu.sync_copy(ep_v.at[pl.ds(0, SLICE)], acc.at[pl.ds(nbase, SLICE)])
    plsc.subcore_barrier()

    def _drain_scatters():
        pltpu.make_async_copy(dist_hbm.at[pl.ds(0, R)], dist_v, semS).wait()

    def chunk_body(ci, carry):
        g = wid + ci * NW

        @pl.when(g < CHUNKS)
        def _():
            rb = g * R

            @pl.when(ci > 0)
            def _():
                pltpu.async_copy(dst_hbm.at[pl.ds(rb * 128, R * 128)],
                                 dst_v, semI)
                pltpu.async_copy(dist_hbm.at[pl.ds(rb, R)], dist_v, semI)
                pltpu.async_copy(sw_hbm.at[pl.ds(rb, R)], sw_v, semI)
                _drain_scatters()
                pltpu.async_copy(src_hbm.at[pl.ds(rb, R)], src_v, semI)

            pltpu.make_async_copy(dst_hbm.at[pl.ds(rb * 128, R * 128)],
                                  dst_v, semI).wait()
            pltpu.make_async_copy(dist_hbm.at[pl.ds(rb, R)], dist_v, semI).wait()
            pltpu.make_async_copy(sw_hbm.at[pl.ds(rb, R)], sw_v, semI).wait()
            pltpu.make_async_copy(src_hbm.at[pl.ds(rb, R)], src_v, semI).wait()

            for sec in range(R // SEC):
                @plsc.parallel_loop(sec * SEC, (sec + 1) * SEC)
                def row_body(j):
                    for gg in range(128 // L):
                        sl = pl.ds(gg * L, L)
                        isc = src_v[j, sl]
                        idc = dst_v[pl.ds(j * 128 + gg * L, L)]
                        qi = plsc.load_gather(tabQ_v, [isc])
                        qj = plsc.load_gather(tabQ_v, [idc])
                        ai = plsc.load_gather(tabA_v, [isc])
                        aj = plsc.load_gather(tabA_v, [idc])
                        pi = plsc.bitcast(ai, jnp.int32)
                        pj = plsc.bitcast(aj, jnp.int32)
                        zi = (pi & m7).astype(jnp.float32) + one
                        zj = (pj & m7).astype(jnp.float32) + one
                        d = dist_v[j, sl]
                        dai = d * ai
                        daj = d * aj
                        eAi = jnp.exp(dai)
                        eAj = jnp.exp(daj)
                        eBi = jnp.exp(dai * rv)
                        eBj = jnp.exp(daj * rv)
                        ui = zi - qi
                        uj = zj - qj
                        ti = ui * (eBi - one)
                        tj = uj * (eBj - one)
                        core = (zj * ui * eAi + zi * uj * eAj
                                + ti * tj - ui * uj + qi * qj)
                        ep_v[pl.ds(j * 128 + gg * L, L)] = (
                            core * sw_v[j, sl] * sv / d)
                for j in range(sec * SEC, (sec + 1) * SEC):
                    pltpu.async_copy(ep_v.at[pl.ds(j * 128, 128)],
                                     acc.at[src_v.at[j]], semS, add=True)
        return carry

    lax.fori_loop(0, CPW, chunk_body, 0)
    _drain_scatters()

    plsc.subcore_barrier()
    pltpu.sync_copy(acc.at[pl.ds(nbase, SLICE)], ep_v.at[pl.ds(0, SLICE)])
    pltpu.sync_copy(ep_v.at[pl.ds(0, SLICE)],
                    out_hbm.at[pl.ds(c * NPAD + nbase, SLICE)])


def kernel(species, edge_src, edge_dst, distances, switch, charges,
           vdw_radii, d3_vdw_radii, valence_electrons,
           cpA, cpB, gamma, charge_scale, scale):
    N = species.shape[0]
    cpA_ = jnp.abs(cpA).astype(jnp.float32)
    cpB_ = jnp.abs(cpB).astype(jnp.float32)

    spi = species.astype(jnp.int32)
    chf = charges.astype(jnp.float32)
    vdw96 = jnp.pad(vdw_radii.astype(jnp.float32), (0, 96 - vdw_radii.shape[0]),
                    constant_values=1.0)
    val96 = jnp.pad(valence_electrons.astype(jnp.float32),
                    (0, 96 - valence_electrons.shape[0]), constant_values=1.0)
    params = jnp.concatenate([
        jnp.full((L,), cpB_ / cpA_, jnp.float32),
        jnp.full((L,), 0.5 * BOHR * jnp.abs(scale), jnp.float32),
        jnp.full((L,), jnp.abs(charge_scale), jnp.float32),
        jnp.full((L,), -cpA_ / BOHR, jnp.float32),
    ])

    src2 = edge_src.astype(jnp.int32).reshape(ROWS, 128)
    dst1 = edge_dst.astype(jnp.int32)
    dist2 = distances.astype(jnp.float32).reshape(ROWS, 128)
    sw2 = switch.astype(jnp.float32).reshape(ROWS, 128)

    out = _coulomb_sc(spi, chf, src2, dst1, dist2, sw2, params, vdw96, val96)
    return (out[:NPAD] + out[NPAD:])[:N]


# speedup vs baseline: 1.1178x; 1.0287x over previous
import functools

import jax
import jax.numpy as jnp
from jax import lax
from jax.experimental import pallas as pl
from jax.experimental.pallas import tpu as pltpu
from jax.experimental.pallas import tpu_sc as plsc

BOHR = 0.52917721067
N_NODES = 50000
N_EDGES = 3200000

NC = 2
NS = 16
NW = NC * NS
L = 16

NPAD = 50176
SLICE = NPAD // NS
NTAIL = N_NODES - (NS - 1) * SLICE
ROWS = N_EDGES // 128
R = 40
SEC = 20
CHUNKS = ROWS // R
CPW = (CHUNKS + NW - 1) // NW


def _splat_f(v):
    return jnp.full((L,), v, jnp.float32)


def _splat_i(v):
    return jnp.full((L,), v, jnp.int32)


@functools.partial(
    pl.kernel,
    out_type=jax.ShapeDtypeStruct((NC * NPAD,), jnp.float32),
    mesh=plsc.VectorSubcoreMesh(
        core_axis_name="c", subcore_axis_name="s",
        num_cores=NC, num_subcores=NS),
    compiler_params=pltpu.CompilerParams(needs_layout_passes=False),
    scratch_types=[
        pltpu.VMEM((NPAD,), jnp.float32),
        pltpu.VMEM((NPAD,), jnp.float32),
        pltpu.VMEM((R, 128), jnp.int32),
        pltpu.VMEM((R * 128,), jnp.int32),
        pltpu.VMEM((R, 128), jnp.float32),
        pltpu.VMEM((R, 128), jnp.float32),
        pltpu.VMEM((R * 128,), jnp.float32),
        pltpu.VMEM((4 * L,), jnp.float32),
        pltpu.VMEM((96,), jnp.float32),
        pltpu.VMEM((96,), jnp.float32),
        pltpu.SemaphoreType.DMA,
        pltpu.SemaphoreType.DMA,
        pltpu.VMEM_SHARED((NPAD,), jnp.float32),
    ],
)
def _coulomb_sc(sp_hbm, ch_hbm, src_hbm, dst_hbm, dist_hbm, sw_hbm,
                par_hbm, vdw_hbm, val_hbm, out_hbm,
                tabQ_v, tabA_v, src_v, dst_v, dist_v, sw_v, ep_v,
                par_v, vdw_v, val_v, semI, semS, acc):
    c = lax.axis_index("c")
    s = lax.axis_index("s")
    wid = s * NC + c
    nbase = s * SLICE

    pltpu.sync_copy(par_hbm, par_v)
    pltpu.sync_copy(vdw_hbm, vdw_v)
    pltpu.sync_copy(val_hbm, val_v)

    rb0 = wid * R
    pltpu.async_copy(src_hbm.at[pl.ds(rb0, R)], src_v, semI)
    pltpu.async_copy(dist_hbm.at[pl.ds(rb0, R)], dist_v, semI)
    pltpu.async_copy(sw_hbm.at[pl.ds(rb0, R)], sw_v, semI)

    rv = par_v[pl.ds(0 * L, L)]
    sv = par_v[pl.ds(1 * L, L)]
    cscv = par_v[pl.ds(2 * L, L)]
    acpv = par_v[pl.ds(3 * L, L)]
    one = _splat_f(1.0)
    m7 = _splat_i(7)
    mm8 = _splat_i(-8)

    @pl.when(s < NS - 1)
    def _():
        pltpu.sync_copy(ch_hbm.at[pl.ds(nbase, SLICE)],
                        ep_v.at[pl.ds(0, SLICE)])

    @pl.when(s == NS - 1)
    def _():
        pltpu.sync_copy(ch_hbm.at[pl.ds(nbase, NTAIL)],
                        ep_v.at[pl.ds(0, NTAIL)])

        def _zt(i, carry):
            ep_v[pl.ds(NTAIL + i * L, L)] = _splat_f(0.0)
            return carry
        lax.fori_loop(0, (SLICE - NTAIL) // L, _zt, 0)

    @plsc.parallel_loop(0, SLICE // L)
    def q_body(i):
        sl = pl.ds(i * L, L)
        tabQ_v[sl] = ep_v[sl] * cscv
    pltpu.sync_copy(tabQ_v.at[pl.ds(0, SLICE)], acc.at[pl.ds(nbase, SLICE)])
    plsc.subcore_barrier()
    rdQ = pltpu.async_copy(acc, tabQ_v, semS)

    @pl.when(s < NS - 1)
    def _():
        pltpu.sync_copy(sp_hbm.at[pl.ds(nbase, SLICE)],
                        dst_v.at[pl.ds(0, SLICE)])

    @pl.when(s == NS - 1)
    def _():
        pltpu.sync_copy(sp_hbm.at[pl.ds(nbase, NTAIL)],
                        dst_v.at[pl.ds(0, NTAIL)])

        def _zt(i, carry):
            dst_v[pl.ds(NTAIL + i * L, L)] = _splat_i(0)
            return carry
        lax.fori_loop(0, (SLICE - NTAIL) // L, _zt, 0)

    @plsc.parallel_loop(0, SLICE // L)
    def a_body(i):
        sl = pl.ds(i * L, L)
        sp = dst_v[sl]
        v = plsc.load_gather(vdw_v, [sp])
        z = plsc.load_gather(val_v, [sp])
        a = acpv / v
        bits = ((lax.bitcast_convert_type(a, jnp.int32) & mm8)
                | (z.astype(jnp.int32) - 1))
        tabA_v[sl] = plsc.bitcast(bits, jnp.float32)
    pltpu.async_copy(dst_hbm.at[pl.ds(rb0 * 128, R * 128)], dst_v, semI)
    rdQ.wait()
    plsc.subcore_barrier()
    pltpu.sync_copy(tabA_v.at[pl.ds(0, SLICE)], acc.at[pl.ds(nbase, SLICE)])
    plsc.subcore_barrier()
    rdA = pltpu.async_copy(acc, tabA_v, semS)

    def _zb(i, carry):
        ep_v[pl.ds(i * L, L)] = _splat_f(0.0)
        return carry
    lax.fori_loop(0, SLICE // L, _zb, 0)
    rdA.wait()
    plsc.subcore_barrier()
    pltpu.sync_copy(ep_v.at[pl.ds(0, SLICE)], acc.at[pl.ds(nbase, SLICE)])
    plsc.subcore_barrier()

    def _drain_scatters():
        pltpu.make_async_copy(dist_hbm.at[pl.ds(0, R)], dist_v, semS).wait()

    def chunk_body(ci, carry):
        g = wid + ci * NW

        @pl.when(g < CHUNKS)
        def _():
            rb = g * R

            @pl.when(ci > 0)
            def _():
                pltpu.async_copy(dst_hbm.at[pl.ds(rb * 128, R * 128)],
                                 dst_v, semI)
                pltpu.async_copy(dist_hbm.at[pl.ds(rb, R)], dist_v, semI)
                pltpu.async_copy(sw_hbm.at[pl.ds(rb, R)], sw_v, semI)
                _drain_scatters()
                pltpu.async_copy(src_hbm.at[pl.ds(rb, R)], src_v, semI)

            pltpu.make_async_copy(ch_hbm.at[pl.ds(0, 4 * R * 128)],
                                  tabQ_v.at[pl.ds(0, 4 * R * 128)],
                                  semI).wait()

            for sec in range(R // SEC):
                @plsc.parallel_loop(sec * SEC, (sec + 1) * SEC)
                def row_body(j):
                    for gg in range(128 // L):
                        sl = pl.ds(gg * L, L)
                        isc = src_v[j, sl]
                        idc = dst_v[pl.ds(j * 128 + gg * L, L)]
                        qi = plsc.load_gather(tabQ_v, [isc])
                        qj = plsc.load_gather(tabQ_v, [idc])
                        ai = plsc.load_gather(tabA_v, [isc])
                        aj = plsc.load_gather(tabA_v, [idc])
                        pi = plsc.bitcast(ai, jnp.int32)
                        pj = plsc.bitcast(aj, jnp.int32)
                        zi = (pi & m7).astype(jnp.float32) + one
                        zj = (pj & m7).astype(jnp.float32) + one
                        d = dist_v[j, sl]
                        dai = d * ai
                        daj = d * aj
                        eAi = jnp.exp(dai)
                        eAj = jnp.exp(daj)
                        eBi = jnp.exp(dai * rv)
                        eBj = jnp.exp(daj * rv)
                        ui = zi - qi
                        uj = zj - qj
                        ti = ui * (eBi - one)
                        tj = uj * (eBj - one)
                        core = (zj * ui * eAi + zi * uj * eAj
                                + ti * tj - ui * uj + qi * qj)
                        ep_v[pl.ds(j * 128 + gg * L, L)] = (
                            core * sw_v[j, sl] / d)
                for j in range(sec * SEC, (sec + 1) * SEC):
                    pltpu.async_copy(ep_v.at[pl.ds(j * 128, 128)],
                                     acc.at[src_v.at[j]], semS, add=True)
        return carry

    lax.fori_loop(0, CPW, chunk_body, 0)
    _drain_scatters()

    plsc.subcore_barrier()
    pltpu.sync_copy(acc.at[pl.ds(nbase, SLICE)], ep_v.at[pl.ds(0, SLICE)])

    @plsc.parallel_loop(0, SLICE // L)
    def fin_body(i):
        sl = pl.ds(i * L, L)
        ep_v[sl] = ep_v[sl] * sv
    pltpu.sync_copy(ep_v.at[pl.ds(0, SLICE)],
                    out_hbm.at[pl.ds(c * NPAD + nbase, SLICE)])


def kernel(species, edge_src, edge_dst, distances, switch, charges,
           vdw_radii, d3_vdw_radii, valence_electrons,
           cpA, cpB, gamma, charge_scale, scale):
    N = species.shape[0]
    cpA_ = jnp.abs(cpA).astype(jnp.float32)
    cpB_ = jnp.abs(cpB).astype(jnp.float32)

    spi = species.astype(jnp.int32)
    chf = charges.astype(jnp.float32)
    vdw96 = jnp.pad(vdw_radii.astype(jnp.float32), (0, 96 - vdw_radii.shape[0]),
                    constant_values=1.0)
    val96 = jnp.pad(valence_electrons.astype(jnp.float32),
                    (0, 96 - valence_electrons.shape[0]), constant_values=1.0)
    params = jnp.concatenate([
        jnp.full((L,), cpB_ / cpA_, jnp.float32),
        jnp.full((L,), 0.5 * BOHR * jnp.abs(scale), jnp.float32),
        jnp.full((L,), jnp.abs(charge_scale), jnp.float32),
        jnp.full((L,), -cpA_ / BOHR, jnp.float32),
    ])

    src2 = edge_src.astype(jnp.int32).reshape(ROWS, 128)
    dst1 = edge_dst.astype(jnp.int32)
    dist2 = distances.astype(jnp.float32).reshape(ROWS, 128)
    sw2 = switch.astype(jnp.float32).reshape(ROWS, 128)

    out = _coulomb_sc(spi, chf, src2, dst1, dist2, sw2, params, vdw96, val96)
    return (out[:NPAD] + out[NPAD:])[:N]


# speedup vs baseline: 1.1414x; 1.0211x over previous
import functools

import jax
import jax.numpy as jnp
from jax import lax
from jax.experimental import pallas as pl
from jax.experimental.pallas import tpu as pltpu
from jax.experimental.pallas import tpu_sc as plsc

BOHR = 0.52917721067
N_NODES = 50000
N_EDGES = 3200000

NC = 2
NS = 16
NW = NC * NS
L = 16

NPAD = 50176
SLICE = NPAD // NS
NTAIL = N_NODES - (NS - 1) * SLICE
ROWS = N_EDGES // 128
R = 40
SEC = 20
CHUNKS = ROWS // R
CPW = (CHUNKS + NW - 1) // NW


def _splat_f(v):
    return jnp.full((L,), v, jnp.float32)


def _splat_i(v):
    return jnp.full((L,), v, jnp.int32)


@functools.partial(
    pl.kernel,
    out_type=jax.ShapeDtypeStruct((NC * NPAD,), jnp.float32),
    mesh=plsc.VectorSubcoreMesh(
        core_axis_name="c", subcore_axis_name="s",
        num_cores=NC, num_subcores=NS),
    compiler_params=pltpu.CompilerParams(needs_layout_passes=False),
    scratch_types=[
        pltpu.VMEM((NPAD,), jnp.float32),
        pltpu.VMEM((NPAD,), jnp.float32),
        pltpu.VMEM((R, 128), jnp.int32),
        pltpu.VMEM((R * 128,), jnp.int32),
        pltpu.VMEM((R, 128), jnp.float32),
        pltpu.VMEM((R, 128), jnp.float32),
        pltpu.VMEM((R * 128,), jnp.float32),
        pltpu.VMEM((4 * L,), jnp.float32),
        pltpu.VMEM((96,), jnp.float32),
        pltpu.VMEM((96,), jnp.float32),
        pltpu.SemaphoreType.DMA,
        pltpu.SemaphoreType.DMA,
        pltpu.VMEM_SHARED((NPAD,), jnp.float32),
    ],
)
def _coulomb_sc(sp_hbm, ch_hbm, src_hbm, dst_hbm, dist_hbm, sw_hbm,
                par_hbm, vdw_hbm, val_hbm, out_hbm,
                tabQ_v, tabA_v, src_v, dst_v, dist_v, sw_v, ep_v,
                par_v, vdw_v, val_v, semI, semS, acc):
    c = lax.axis_index("c")
    s = lax.axis_index("s")
    wid = s * NC + c
    nbase = s * SLICE

    pltpu.async_copy(par_hbm, par_v, semS)
    pltpu.async_copy(vdw_hbm, vdw_v, semS)
    pltpu.async_copy(val_hbm, val_v, semS)

    @pl.when(s < NS - 1)
    def _():
        pltpu.async_copy(ch_hbm.at[pl.ds(nbase, SLICE)],
                         ep_v.at[pl.ds(0, SLICE)], semS)
        pltpu.async_copy(sp_hbm.at[pl.ds(nbase, SLICE)],
                         dst_v.at[pl.ds(0, SLICE)], semS)

    @pl.when(s == NS - 1)
    def _():
        pltpu.async_copy(ch_hbm.at[pl.ds(nbase, NTAIL)],
                         ep_v.at[pl.ds(0, NTAIL)], semS)
        pltpu.async_copy(sp_hbm.at[pl.ds(nbase, NTAIL)],
                         dst_v.at[pl.ds(0, NTAIL)], semS)

    rb0 = wid * R
    pltpu.async_copy(src_hbm.at[pl.ds(rb0, R)], src_v, semI)
    pltpu.async_copy(dist_hbm.at[pl.ds(rb0, R)], dist_v, semI)
    pltpu.async_copy(sw_hbm.at[pl.ds(rb0, R)], sw_v, semI)

    _HDR = (4 * L + 96 + 96)

    @pl.when(s < NS - 1)
    def _():
        pltpu.make_async_copy(ch_hbm.at[pl.ds(0, _HDR + 2 * SLICE)],
                              tabQ_v.at[pl.ds(0, _HDR + 2 * SLICE)],
                              semS).wait()

    @pl.when(s == NS - 1)
    def _():
        pltpu.make_async_copy(ch_hbm.at[pl.ds(0, _HDR + 2 * NTAIL)],
                              tabQ_v.at[pl.ds(0, _HDR + 2 * NTAIL)],
                              semS).wait()

        def _zt(i, carry):
            ep_v[pl.ds(NTAIL + i * L, L)] = _splat_f(0.0)
            dst_v[pl.ds(NTAIL + i * L, L)] = _splat_i(0)
            return carry
        lax.fori_loop(0, (SLICE - NTAIL) // L, _zt, 0)

    rv = par_v[pl.ds(0 * L, L)]
    sv = par_v[pl.ds(1 * L, L)]
    cscv = par_v[pl.ds(2 * L, L)]
    acpv = par_v[pl.ds(3 * L, L)]
    one = _splat_f(1.0)
    m7 = _splat_i(7)
    mm8 = _splat_i(-8)

    @plsc.parallel_loop(0, SLICE // L)
    def q_body(i):
        sl = pl.ds(i * L, L)
        tabQ_v[sl] = ep_v[sl] * cscv
    pltpu.sync_copy(tabQ_v.at[pl.ds(0, SLICE)], acc.at[pl.ds(nbase, SLICE)])
    plsc.subcore_barrier()
    rdQ = pltpu.async_copy(acc, tabQ_v, semS)

    @plsc.parallel_loop(0, SLICE // L)
    def a_body(i):
        sl = pl.ds(i * L, L)
        sp = dst_v[sl]
        v = plsc.load_gather(vdw_v, [sp])
        z = plsc.load_gather(val_v, [sp])
        a = acpv / v
        bits = ((lax.bitcast_convert_type(a, jnp.int32) & mm8)
                | (z.astype(jnp.int32) - 1))
        tabA_v[sl] = plsc.bitcast(bits, jnp.float32)
    pltpu.async_copy(dst_hbm.at[pl.ds(rb0 * 128, R * 128)], dst_v, semI)
    rdQ.wait()
    plsc.subcore_barrier()
    pltpu.sync_copy(tabA_v.at[pl.ds(0, SLICE)], acc.at[pl.ds(nbase, SLICE)])
    plsc.subcore_barrier()
    rdA = pltpu.async_copy(acc, tabA_v, semS)

    def _zb(i, carry):
        ep_v[pl.ds(i * L, L)] = _splat_f(0.0)
        return carry
    lax.fori_loop(0, SLICE // L, _zb, 0)
    rdA.wait()
    plsc.subcore_barrier()
    pltpu.sync_copy(ep_v.at[pl.ds(0, SLICE)], acc.at[pl.ds(nbase, SLICE)])
    plsc.subcore_barrier()

    def _drain_scatters():
        pltpu.make_async_copy(dist_hbm.at[pl.ds(0, R)], dist_v, semS).wait()

    def chunk_body(ci, carry):
        g = wid + ci * NW

        @pl.when(g < CHUNKS)
        def _():
            rb = g * R

            @pl.when(ci > 0)
            def _():
                pltpu.async_copy(dst_hbm.at[pl.ds(rb * 128, R * 128)],
                                 dst_v, semI)
                pltpu.async_copy(dist_hbm.at[pl.ds(rb, R)], dist_v, semI)
                pltpu.async_copy(sw_hbm.at[pl.ds(rb, R)], sw_v, semI)
                _drain_scatters()
                pltpu.async_copy(src_hbm.at[pl.ds(rb, R)], src_v, semI)

            pltpu.make_async_copy(ch_hbm.at[pl.ds(0, 4 * R * 128)],
                                  tabQ_v.at[pl.ds(0, 4 * R * 128)],
                                  semI).wait()

            for sec in range(R // SEC):
                @plsc.parallel_loop(sec * SEC, (sec + 1) * SEC)
                def row_body(j):
                    for gg in range(128 // L):
                        sl = pl.ds(gg * L, L)
                        isc = src_v[j, sl]
                        idc = dst_v[pl.ds(j * 128 + gg * L, L)]
                        qi = plsc.load_gather(tabQ_v, [isc])
                        qj = plsc.load_gather(tabQ_v, [idc])
                        ai = plsc.load_gather(tabA_v, [isc])
                        aj = plsc.load_gather(tabA_v, [idc])
                        pi = plsc.bitcast(ai, jnp.int32)
                        pj = plsc.bitcast(aj, jnp.int32)
                        zi = (pi & m7).astype(jnp.float32) + one
                        zj = (pj & m7).astype(jnp.float32) + one
                        d = dist_v[j, sl]
                        dai = d * ai
                        daj = d * aj
                        eAi = jnp.exp(dai)
                        eAj = jnp.exp(daj)
                        eBi = jnp.exp(dai * rv)
                        eBj = jnp.exp(daj * rv)
                        ui = zi - qi
                        uj = zj - qj
                        ti = ui * (eBi - one)
                        tj = uj * (eBj - one)
                        core = (zj * ui * eAi + zi * uj * eAj
                                + ti * tj - ui * uj + qi * qj)
                        ep_v[pl.ds(j * 128 + gg * L, L)] = (
                            core * sw_v[j, sl] / d)
                for j in range(sec * SEC, (sec + 1) * SEC):
                    pltpu.async_copy(ep_v.at[pl.ds(j * 128, 128)],
                                     acc.at[src_v.at[j]], semS, add=True)
        return carry

    lax.fori_loop(0, CPW, chunk_body, 0)
    _drain_scatters()

    plsc.subcore_barrier()
    pltpu.sync_copy(acc.at[pl.ds(nbase, SLICE)], ep_v.at[pl.ds(0, SLICE)])

    @plsc.parallel_loop(0, SLICE // L)
    def fin_body(i):
        sl = pl.ds(i * L, L)
        ep_v[sl] = ep_v[sl] * sv
    pltpu.sync_copy(ep_v.at[pl.ds(0, SLICE)],
                    out_hbm.at[pl.ds(c * NPAD + nbase, SLICE)])


def kernel(species, edge_src, edge_dst, distances, switch, charges,
           vdw_radii, d3_vdw_radii, valence_electrons,
           cpA, cpB, gamma, charge_scale, scale):
    N = species.shape[0]
    cpA_ = jnp.abs(cpA).astype(jnp.float32)
    cpB_ = jnp.abs(cpB).astype(jnp.float32)

    spi = species.astype(jnp.int32)
    chf = charges.astype(jnp.float32)
    vdw96 = jnp.pad(vdw_radii.astype(jnp.float32), (0, 96 - vdw_radii.shape[0]),
                    constant_values=1.0)
    val96 = jnp.pad(valence_electrons.astype(jnp.float32),
                    (0, 96 - valence_electrons.shape[0]), constant_values=1.0)
    params = jnp.concatenate([
        jnp.full((L,), cpB_ / cpA_, jnp.float32),
        jnp.full((L,), 0.5 * BOHR * jnp.abs(scale), jnp.float32),
        jnp.full((L,), jnp.abs(charge_scale), jnp.float32),
        jnp.full((L,), -cpA_ / BOHR, jnp.float32),
    ])

    src2 = edge_src.astype(jnp.int32).reshape(ROWS, 128)
    dst1 = edge_dst.astype(jnp.int32)
    dist2 = distances.astype(jnp.float32).reshape(ROWS, 128)
    sw2 = switch.astype(jnp.float32).reshape(ROWS, 128)

    out = _coulomb_sc(spi, chf, src2, dst1, dist2, sw2, params, vdw96, val96)
    return (out[:NPAD] + out[NPAD:])[:N]
